# baseline (device time: 67982 ns/iter reference)
import jax
import jax.numpy as jnp
from jax import lax
from jax.experimental import pallas as pl
from jax.experimental.pallas import tpu as pltpu

NZ = 4
T = 256
QB = T // 4
FB = QB // 2
D = 4096
N_FULL = NZ * D
NR = NZ - 1
S = 4
SD = D // S

J_ORDER = (1, 3, 2)


def kernel(x, W):
    def body(x_ref, w_ref, out_ref, g_ref,
             zs, zr, xds, xdr, yds, ydr, xfs, xfr, yfs, yfr):
        my_x = lax.axis_index("x")
        my_y = lax.axis_index("y")
        my_z = lax.axis_index("z")
        r = 2 * my_x + my_y
        r_x = 2 * (1 - my_x) + my_y
        r_y = 2 * my_x + (1 - my_y)

        barrier_sem = pltpu.get_barrier_semaphore()
        for k in range(1, NZ):
            pl.semaphore_signal(
                barrier_sem, inc=1,
                device_id=(my_x, my_y, (my_z + k) % NZ),
                device_id_type=pl.DeviceIdType.MESH,
            )
        for dev in ((1 - my_x, my_y, my_z), (my_x, 1 - my_y, my_z)):
            pl.semaphore_signal(
                barrier_sem, inc=1,
                device_id=dev, device_id_type=pl.DeviceIdType.MESH,
            )
        pl.semaphore_wait(barrier_sem, NR + 2)

        def desc(src, dst, send_sem, recv_sem, dev):
            return pltpu.make_async_remote_copy(
                src_ref=src, dst_ref=dst,
                send_sem=send_sem, recv_sem=recv_sem,
                device_id=dev, device_id_type=pl.DeviceIdType.MESH,
            )

        x_nbr = (1 - my_x, my_y, my_z)
        y_nbr = (my_x, 1 - my_y, my_z)

        def zsend_desc(k, s):
            blk = g_ref.at[my_z, s, r]
            return desc(blk, blk, zs.at[k - 1, s], zr.at[k - 1, s],
                        (my_x, my_y, (my_z + k) % NZ))

        def zrecv_desc(j, s):
            c = (my_z + j) % NZ
            blk = g_ref.at[c, s, r]
            return desc(blk, blk, zs.at[3 - j, s], zr.at[3 - j, s],
                        (my_x, my_y, c))

        def xdir_desc(idx, j, s):
            c = (my_z + j) % NZ
            blk = g_ref.at[c, s, r]
            return desc(blk, blk, xds.at[idx, s], xdr.at[idx, s], x_nbr)

        def ydir_desc(idx, j, s):
            c = (my_z + j) % NZ
            blk = g_ref.at[c, s, r]
            return desc(blk, blk, yds.at[idx, s], ydr.at[idx, s], y_nbr)

        def xfwd_desc(idx, j, s):
            c = (my_z + j) % NZ
            blk = g_ref.at[c, s, r_y, pl.ds(0, FB)]
            return desc(blk, blk, xfs.at[idx, s], xfr.at[idx, s], x_nbr)

        def yfwd_desc(idx, j, s):
            c = (my_z + j) % NZ
            blk = g_ref.at[c, s, r_x, pl.ds(FB, FB)]
            return desc(blk, blk, yfs.at[idx, s], yfr.at[idx, s], y_nbr)

        x_bf = x_ref[...].astype(jnp.bfloat16)
        zsend = {}
        for s in range(S):
            logits = jnp.dot(x_bf,
                             w_ref[:, s * SD:(s + 1) * SD].astype(
                                 jnp.bfloat16),
                             preferred_element_type=jnp.float32)
            e_s = jnp.exp(logits).astype(jnp.bfloat16)
            for q in range(4):
                g_ref[my_z, s, q] = e_s[q * QB:(q + 1) * QB, :]
            for k in range(1, NZ):
                zsend[(k, s)] = zsend_desc(k, s)
                zsend[(k, s)].start()

        zrecv, xdir, ydir, xfwd, yfwd = {}, {}, {}, {}, {}
        for idx, j in enumerate(J_ORDER):
            for s in range(S):
                zrecv[(idx, s)] = zrecv_desc(j, s)
                zrecv[(idx, s)].wait_recv()
                xdir[(idx, s)] = xdir_desc(idx, j, s)
                xdir[(idx, s)].start()
                ydir[(idx, s)] = ydir_desc(idx, j, s)
                ydir[(idx, s)].start()
            if idx >= 1:
                pj = J_ORDER[idx - 1]
                for s in range(S):
                    xdir[(idx - 1, s)].wait_recv()
                    yfwd[(idx - 1, s)] = yfwd_desc(idx - 1, pj, s)
                    yfwd[(idx - 1, s)].start()
                    ydir[(idx - 1, s)].wait_recv()
                    xfwd[(idx - 1, s)] = xfwd_desc(idx - 1, pj, s)
                    xfwd[(idx - 1, s)].start()

        lj = J_ORDER[-1]
        for s in range(S):
            xdir[(2, s)].wait_recv()
            yfwd[(2, s)] = yfwd_desc(2, lj, s)
            yfwd[(2, s)].start()
            ydir[(2, s)].wait_recv()
            xfwd[(2, s)] = xfwd_desc(2, lj, s)
            xfwd[(2, s)].start()

        for idx in range(NR):
            for s in range(S):
                xfwd[(idx, s)].wait_recv()
                yfwd[(idx, s)].wait_recv()

        for ds in (zsend, xdir, ydir, xfwd, yfwd):
            for d in ds.values():
                d.wait_send()

        accs = []
        for q in range(4):
            aq = jnp.zeros((QB, 1), jnp.float32)
            for c in range(NZ):
                for s in range(S):
                    aq = aq + jnp.sum(
                        g_ref[c, s, q].astype(jnp.float32),
                        axis=1, keepdims=True)
            accs.append(aq)
        inv = 1.0 / jnp.concatenate(accs, axis=0)

        for c in range(NZ):
            for s in range(S):
                for q in range(4):
                    out_ref[q * QB:(q + 1) * QB,
                            c * D + s * SD:(c * D + (s + 1) * SD)] = (
                        g_ref[c, s, q].astype(jnp.float32)
                        * inv[q * QB:(q + 1) * QB])

    return pl.pallas_call(
        body,
        out_shape=jax.ShapeDtypeStruct((T, N_FULL), jnp.float32),
        in_specs=[
            pl.BlockSpec(memory_space=pltpu.VMEM),
            pl.BlockSpec(memory_space=pltpu.VMEM),
        ],
        out_specs=pl.BlockSpec(memory_space=pltpu.VMEM),
        scratch_shapes=[pltpu.VMEM((NZ, S, 4, QB, SD), jnp.bfloat16)]
        + [pltpu.SemaphoreType.DMA((NR, S))] * 10,
        compiler_params=pltpu.CompilerParams(collective_id=0),
    )(x, W)


# device time: 66798 ns/iter; 1.0177x vs baseline; 1.0177x over previous
import jax
import jax.numpy as jnp
from jax import lax
from jax.experimental import pallas as pl
from jax.experimental.pallas import tpu as pltpu

NZ = 4
T = 256
QB = T // 4
FB = QB // 2
D = 4096
N_FULL = NZ * D
NH = NZ - 1
S = 4
SD = D // S


def kernel(x, W):
    def body(x_ref, w_ref, out_ref, g_ref,
             zs, zr, xds, xdr, yds, ydr, xfs, xfr, yfs, yfr):
        my_x = lax.axis_index("x")
        my_y = lax.axis_index("y")
        my_z = lax.axis_index("z")
        left = (my_z - 1) % NZ
        right = (my_z + 1) % NZ
        r = 2 * my_x + my_y
        r_x = 2 * (1 - my_x) + my_y
        r_y = 2 * my_x + (1 - my_y)

        barrier_sem = pltpu.get_barrier_semaphore()
        for dev in ((my_x, my_y, left), (my_x, my_y, right),
                    (1 - my_x, my_y, my_z), (my_x, 1 - my_y, my_z)):
            pl.semaphore_signal(
                barrier_sem, inc=1,
                device_id=dev, device_id_type=pl.DeviceIdType.MESH,
            )
        pl.semaphore_wait(barrier_sem, 4)

        def desc(src, dst, send_sem, recv_sem, dev):
            return pltpu.make_async_remote_copy(
                src_ref=src, dst_ref=dst,
                send_sem=send_sem, recv_sem=recv_sem,
                device_id=dev, device_id_type=pl.DeviceIdType.MESH,
            )

        x_nbr = (1 - my_x, my_y, my_z)
        y_nbr = (my_x, 1 - my_y, my_z)

        def ring0_desc(s):
            blk = g_ref.at[my_z, s, r]
            return desc(blk, blk, zs.at[0, s], zr.at[0, s],
                        (my_x, my_y, right))

        def ring_desc(h):
            c = (my_z - h) % NZ
            blk = g_ref.at[c, :, r]
            return desc(blk, blk, zs.at[h, 0], zr.at[h, 0],
                        (my_x, my_y, right))

        def xdir_desc(h):
            c = (my_z - h - 1) % NZ
            blk = g_ref.at[c, :, r]
            return desc(blk, blk, xds.at[h, 0], xdr.at[h, 0], x_nbr)

        def ydir_desc(h):
            c = (my_z - h - 1) % NZ
            blk = g_ref.at[c, :, r]
            return desc(blk, blk, yds.at[h, 0], ydr.at[h, 0], y_nbr)

        def xfwd_desc(h):
            c = (my_z - h - 1) % NZ
            blk = g_ref.at[c, :, r_y, pl.ds(0, FB)]
            return desc(blk, blk, xfs.at[h, 0], xfr.at[h, 0], x_nbr)

        def yfwd_desc(h):
            c = (my_z - h - 1) % NZ
            blk = g_ref.at[c, :, r_x, pl.ds(FB, FB)]
            return desc(blk, blk, yfs.at[h, 0], yfr.at[h, 0], y_nbr)

        def chunk_sums(c):
            return [
                sum(jnp.sum(g_ref[c, s, q].astype(jnp.float32),
                            axis=1, keepdims=True) for s in range(S))
                for q in range(4)
            ]

        x_bf = x_ref[...].astype(jnp.bfloat16)
        ring0 = {}
        for s in range(S):
            logits = jnp.dot(x_bf,
                             w_ref[:, s * SD:(s + 1) * SD].astype(
                                 jnp.bfloat16),
                             preferred_element_type=jnp.float32)
            e_s = jnp.exp(logits).astype(jnp.bfloat16)
            for q in range(4):
                g_ref[my_z, s, q] = e_s[q * QB:(q + 1) * QB, :]
            ring0[s] = ring0_desc(s)
            ring0[s].start()

        for s in range(S):
            ring0[s].wait_recv()
        ring1 = ring_desc(1)
        ring1.start()
        xdir0 = xdir_desc(0)
        xdir0.start()
        ydir0 = ydir_desc(0)
        ydir0.start()

        ring1.wait_recv()
        ring2 = ring_desc(2)
        ring2.start()
        xdir1 = xdir_desc(1)
        xdir1.start()
        ydir1 = ydir_desc(1)
        ydir1.start()
        xdir0.wait_recv()
        yfwd0 = yfwd_desc(0)
        yfwd0.start()
        ydir0.wait_recv()
        xfwd0 = xfwd_desc(0)
        xfwd0.start()

        ring2.wait_recv()
        xdir2 = xdir_desc(2)
        xdir2.start()
        ydir2 = ydir_desc(2)
        ydir2.start()
        xdir1.wait_recv()
        yfwd1 = yfwd_desc(1)
        yfwd1.start()
        ydir1.wait_recv()
        xfwd1 = xfwd_desc(1)
        xfwd1.start()

        acc_q = chunk_sums(my_z)

        xfwd0.wait_recv()
        yfwd0.wait_recv()
        c0 = (my_z - 1) % NZ
        acc_q = [a + b for a, b in zip(acc_q, chunk_sums(c0))]

        xdir2.wait_recv()
        yfwd2 = yfwd_desc(2)
        yfwd2.start()
        ydir2.wait_recv()
        xfwd2 = xfwd_desc(2)
        xfwd2.start()

        xfwd1.wait_recv()
        yfwd1.wait_recv()
        c1 = (my_z - 2) % NZ
        acc_q = [a + b for a, b in zip(acc_q, chunk_sums(c1))]

        xfwd2.wait_recv()
        yfwd2.wait_recv()
        c2 = (my_z - 3) % NZ
        acc_q = [a + b for a, b in zip(acc_q, chunk_sums(c2))]

        for d in (list(ring0.values())
                  + [ring1, ring2, xdir0, xdir1, xdir2,
                     ydir0, ydir1, ydir2, xfwd0, xfwd1, xfwd2,
                     yfwd0, yfwd1, yfwd2]):
            d.wait_send()

        inv = 1.0 / jnp.concatenate(acc_q, axis=0)

        for c in range(NZ):
            for s in range(S):
                for q in range(4):
                    out_ref[q * QB:(q + 1) * QB,
                            c * D + s * SD:(c * D + (s + 1) * SD)] = (
                        g_ref[c, s, q].astype(jnp.float32)
                        * inv[q * QB:(q + 1) * QB])

    return pl.pallas_call(
        body,
        out_shape=jax.ShapeDtypeStruct((T, N_FULL), jnp.float32),
        in_specs=[
            pl.BlockSpec(memory_space=pltpu.VMEM),
            pl.BlockSpec(memory_space=pltpu.VMEM),
        ],
        out_specs=pl.BlockSpec(memory_space=pltpu.VMEM),
        scratch_shapes=[pltpu.VMEM((NZ, S, 4, QB, SD), jnp.bfloat16)]
        + [pltpu.SemaphoreType.DMA((NH, S))] * 10,
        compiler_params=pltpu.CompilerParams(collective_id=0),
    )(x, W)


# device time: 61707 ns/iter; 1.1017x vs baseline; 1.0825x over previous
import jax
import jax.numpy as jnp
from jax import lax
from jax.experimental import pallas as pl
from jax.experimental.pallas import tpu as pltpu

NZ = 4
T = 256
QB = T // 4
FB = QB // 2
D = 4096
N_FULL = NZ * D
NH = NZ - 1
S = 8
SD = D // S


def kernel(x, W):
    def body(x_ref, w_ref, out_ref, g_ref,
             zs, zr, xds, xdr, yds, ydr, xfs, xfr, yfs, yfr):
        my_x = lax.axis_index("x")
        my_y = lax.axis_index("y")
        my_z = lax.axis_index("z")
        left = (my_z - 1) % NZ
        right = (my_z + 1) % NZ
        r = 2 * my_x + my_y
        r_x = 2 * (1 - my_x) + my_y
        r_y = 2 * my_x + (1 - my_y)

        barrier_sem = pltpu.get_barrier_semaphore()
        for dev in ((my_x, my_y, left), (my_x, my_y, right),
                    (1 - my_x, my_y, my_z), (my_x, 1 - my_y, my_z)):
            pl.semaphore_signal(
                barrier_sem, inc=1,
                device_id=dev, device_id_type=pl.DeviceIdType.MESH,
            )
        pl.semaphore_wait(barrier_sem, 4)

        def desc(rows, nrows, cols, send_sem, recv_sem, dev):
            return pltpu.make_async_remote_copy(
                src_ref=g_ref.at[pl.ds(rows, nrows), pl.ds(cols, SD)],
                dst_ref=g_ref.at[pl.ds(rows, nrows), pl.ds(cols, SD)],
                send_sem=send_sem,
                recv_sem=recv_sem,
                device_id=dev,
                device_id_type=pl.DeviceIdType.MESH,
            )

        x_nbr = (1 - my_x, my_y, my_z)
        y_nbr = (my_x, 1 - my_y, my_z)

        def ring_desc(h, s):
            c = (my_z - h) % NZ
            return desc(r * QB, QB, c * D + s * SD, zs.at[h, s],
                        zr.at[h, s], (my_x, my_y, right))

        def xdir_desc(h, s):
            c = (my_z - h - 1) % NZ
            return desc(r * QB, QB, c * D + s * SD, xds.at[h, s],
                        xdr.at[h, s], x_nbr)

        def ydir_desc(h, s):
            c = (my_z - h - 1) % NZ
            return desc(r * QB, QB, c * D + s * SD, yds.at[h, s],
                        ydr.at[h, s], y_nbr)

        def xfwd_desc(h, s):
            c = (my_z - h - 1) % NZ
            return desc(r_y * QB, FB, c * D + s * SD, xfs.at[h, s],
                        xfr.at[h, s], x_nbr)

        def yfwd_desc(h, s):
            c = (my_z - h - 1) % NZ
            return desc(r_x * QB + FB, FB, c * D + s * SD, yfs.at[h, s],
                        yfr.at[h, s], y_nbr)

        x_bf = x_ref[...].astype(jnp.bfloat16)
        ring = {}
        for s in range(S):
            logits = jnp.dot(x_bf,
                             w_ref[:, s * SD:(s + 1) * SD].astype(
                                 jnp.bfloat16),
                             preferred_element_type=jnp.float32)
            g_ref[:, pl.ds(my_z * D + s * SD, SD)] = (
                jnp.exp(logits).astype(jnp.bfloat16))
            ring[(0, s)] = ring_desc(0, s)
            ring[(0, s)].start()

        xdir, ydir, xfwd, yfwd = {}, {}, {}, {}
        for h in range(NH):
            for s in range(S):
                ring[(h, s)].wait_recv()
                if h + 1 < NH:
                    ring[(h + 1, s)] = ring_desc(h + 1, s)
                    ring[(h + 1, s)].start()
                xdir[(h, s)] = xdir_desc(h, s)
                xdir[(h, s)].start()
                ydir[(h, s)] = ydir_desc(h, s)
                ydir[(h, s)].start()
                if h >= 1:
                    xdir[(h - 1, s)].wait_recv()
                    yfwd[(h - 1, s)] = yfwd_desc(h - 1, s)
                    yfwd[(h - 1, s)].start()
                    ydir[(h - 1, s)].wait_recv()
                    xfwd[(h - 1, s)] = xfwd_desc(h - 1, s)
                    xfwd[(h - 1, s)].start()

        for s in range(S):
            xdir[(NH - 1, s)].wait_recv()
            yfwd[(NH - 1, s)] = yfwd_desc(NH - 1, s)
            yfwd[(NH - 1, s)].start()
            ydir[(NH - 1, s)].wait_recv()
            xfwd[(NH - 1, s)] = xfwd_desc(NH - 1, s)
            xfwd[(NH - 1, s)].start()

        acc = jnp.sum(
            g_ref[:, pl.ds(my_z * D, D)].astype(jnp.float32),
            axis=1, keepdims=True)
        for h in range(NH):
            for s in range(S):
                xfwd[(h, s)].wait_recv()
                yfwd[(h, s)].wait_recv()
            c = (my_z - h - 1) % NZ
            acc = acc + jnp.sum(
                g_ref[:, pl.ds(c * D, D)].astype(jnp.float32),
                axis=1, keepdims=True)

        for ds in (ring, xdir, ydir, xfwd, yfwd):
            for d in ds.values():
                d.wait_send()
        inv = 1.0 / acc
        for c in range(NZ):
            out_ref[:, c * D:(c + 1) * D] = (
                g_ref[:, c * D:(c + 1) * D].astype(jnp.float32) * inv)

    return pl.pallas_call(
        body,
        out_shape=jax.ShapeDtypeStruct((T, N_FULL), jnp.float32),
        in_specs=[
            pl.BlockSpec(memory_space=pltpu.VMEM),
            pl.BlockSpec(memory_space=pltpu.VMEM),
        ],
        out_specs=pl.BlockSpec(memory_space=pltpu.VMEM),
        scratch_shapes=[pltpu.VMEM((T, N_FULL), jnp.bfloat16)]
        + [pltpu.SemaphoreType.DMA((NH, S))] * 10,
        compiler_params=pltpu.CompilerParams(collective_id=0),
    )(x, W)


# device time: 61364 ns/iter; 1.1078x vs baseline; 1.0056x over previous
import jax
import jax.numpy as jnp
from jax import lax
from jax.experimental import pallas as pl
from jax.experimental.pallas import tpu as pltpu

NZ = 4
T = 256
QB = T // 4
FB = QB // 2
D = 4096
N_FULL = NZ * D
NH = NZ - 1
S = 4
SD = D // S


def kernel(x, W):
    def body(x_ref, w_ref, out_ref, g_ref,
             zs, zr, xds, xdr, yds, ydr, xfs, xfr, yfs, yfr):
        my_x = lax.axis_index("x")
        my_y = lax.axis_index("y")
        my_z = lax.axis_index("z")
        left = (my_z - 1) % NZ
        right = (my_z + 1) % NZ
        r = 2 * my_x + my_y
        r_x = 2 * (1 - my_x) + my_y
        r_y = 2 * my_x + (1 - my_y)

        barrier_sem = pltpu.get_barrier_semaphore()
        for dev in ((my_x, my_y, left), (my_x, my_y, right),
                    (1 - my_x, my_y, my_z), (my_x, 1 - my_y, my_z)):
            pl.semaphore_signal(
                barrier_sem, inc=1,
                device_id=dev, device_id_type=pl.DeviceIdType.MESH,
            )
        pl.semaphore_wait(barrier_sem, 4)

        def desc(rows, nrows, cols, send_sem, recv_sem, dev):
            return pltpu.make_async_remote_copy(
                src_ref=g_ref.at[pl.ds(rows, nrows), pl.ds(cols, SD)],
                dst_ref=g_ref.at[pl.ds(rows, nrows), pl.ds(cols, SD)],
                send_sem=send_sem,
                recv_sem=recv_sem,
                device_id=dev,
                device_id_type=pl.DeviceIdType.MESH,
            )

        x_nbr = (1 - my_x, my_y, my_z)
        y_nbr = (my_x, 1 - my_y, my_z)

        def ring_desc(h, s):
            c = (my_z - h) % NZ
            return desc(r * QB, QB, c * D + s * SD, zs.at[h, s],
                        zr.at[h, s], (my_x, my_y, right))

        def xdir_desc(h, s):
            c = (my_z - h - 1) % NZ
            return desc(r * QB, QB, c * D + s * SD, xds.at[h, s],
                        xdr.at[h, s], x_nbr)

        def ydir_desc(h, s):
            c = (my_z - h - 1) % NZ
            return desc(r * QB, QB, c * D + s * SD, yds.at[h, s],
                        ydr.at[h, s], y_nbr)

        def xfwd_desc(h, s):
            c = (my_z - h - 1) % NZ
            return desc(r_y * QB, FB, c * D + s * SD, xfs.at[h, s],
                        xfr.at[h, s], x_nbr)

        def yfwd_desc(h, s):
            c = (my_z - h - 1) % NZ
            return desc(r_x * QB + FB, FB, c * D + s * SD, yfs.at[h, s],
                        yfr.at[h, s], y_nbr)

        x_bf = x_ref[...].astype(jnp.bfloat16)
        ring = {}
        for s in range(S):
            logits = jnp.dot(x_bf,
                             w_ref[:, s * SD:(s + 1) * SD].astype(
                                 jnp.bfloat16),
                             preferred_element_type=jnp.float32)
            g_ref[:, pl.ds(my_z * D + s * SD, SD)] = (
                jnp.exp(logits).astype(jnp.bfloat16))
            ring[(0, s)] = ring_desc(0, s)
            ring[(0, s)].start()

        xdir, ydir, xfwd, yfwd = {}, {}, {}, {}
        for h in range(NH):
            for s in range(S):
                ring[(h, s)].wait_recv()
                if h + 1 < NH:
                    ring[(h + 1, s)] = ring_desc(h + 1, s)
                    ring[(h + 1, s)].start()
                xdir[(h, s)] = xdir_desc(h, s)
                xdir[(h, s)].start()
                ydir[(h, s)] = ydir_desc(h, s)
                ydir[(h, s)].start()
                if h >= 1:
                    xdir[(h - 1, s)].wait_recv()
                    yfwd[(h - 1, s)] = yfwd_desc(h - 1, s)
                    yfwd[(h - 1, s)].start()
                    ydir[(h - 1, s)].wait_recv()
                    xfwd[(h - 1, s)] = xfwd_desc(h - 1, s)
                    xfwd[(h - 1, s)].start()

        for s in range(S):
            xdir[(NH - 1, s)].wait_recv()
            yfwd[(NH - 1, s)] = yfwd_desc(NH - 1, s)
            yfwd[(NH - 1, s)].start()
            ydir[(NH - 1, s)].wait_recv()
            xfwd[(NH - 1, s)] = xfwd_desc(NH - 1, s)
            xfwd[(NH - 1, s)].start()

        acc = jnp.sum(
            g_ref[:, pl.ds(my_z * D, D)].astype(jnp.float32),
            axis=1, keepdims=True)
        for h in range(NH):
            for s in range(S):
                xfwd[(h, s)].wait_recv()
                yfwd[(h, s)].wait_recv()
            c = (my_z - h - 1) % NZ
            acc = acc + jnp.sum(
                g_ref[:, pl.ds(c * D, D)].astype(jnp.float32),
                axis=1, keepdims=True)

        for ds in (ring, xdir, ydir, xfwd, yfwd):
            for d in ds.values():
                d.wait_send()
        inv = 1.0 / acc
        for c in range(NZ):
            out_ref[:, c * D:(c + 1) * D] = (
                g_ref[:, c * D:(c + 1) * D].astype(jnp.float32) * inv)

    return pl.pallas_call(
        body,
        out_shape=jax.ShapeDtypeStruct((T, N_FULL), jnp.float32),
        in_specs=[
            pl.BlockSpec(memory_space=pltpu.VMEM),
            pl.BlockSpec(memory_space=pltpu.VMEM),
        ],
        out_specs=pl.BlockSpec(memory_space=pltpu.VMEM),
        scratch_shapes=[pltpu.VMEM((T, N_FULL), jnp.bfloat16)]
        + [pltpu.SemaphoreType.DMA((NH, S))] * 10,
        compiler_params=pltpu.CompilerParams(collective_id=0),
    )(x, W)
